# Initial kernel scaffold; baseline (speedup 1.0000x reference)
import functools

import jax
import jax.numpy as jnp
from jax import lax
from jax.experimental import pallas as pl
from jax.experimental.pallas import tpu as pltpu
from jax.experimental.pallas import tpu_sc as plsc

NC = 2
NS = 16
NW = NC * NS
LANES = 16
CH = 80


def _sc_degree(dst, n):
  e = dst.shape[0]
  epw = e // NW
  mesh = plsc.VectorSubcoreMesh(core_axis_name="c", subcore_axis_name="s")

  @functools.partial(
      pl.kernel,
      mesh=mesh,
      out_type=jax.ShapeDtypeStruct((NW, n), jnp.float32),
      scratch_types=[
          pltpu.VMEM((n,), jnp.float32),
          pltpu.VMEM((epw,), jnp.int32),
      ],
  )
  def deg_kernel(dst_hbm, out_hbm, degbuf, dstbuf):
    cid = lax.axis_index("c")
    sid = lax.axis_index("s")
    wid = sid * NC + cid

    zero = jnp.zeros((LANES,), jnp.float32)

    def zbody(i, carry):
      degbuf[pl.ds(i * LANES, LANES)] = zero
      return carry

    lax.fori_loop(0, n // LANES, zbody, 0)

    pltpu.sync_copy(dst_hbm.at[pl.ds(wid * epw, epw)], dstbuf)

    ones = jnp.ones((LANES,), jnp.float32)

    def body(j, carry):
      idx = dstbuf[pl.ds(j * LANES, LANES)]
      plsc.addupdate_scatter(degbuf, [idx], ones)
      return carry

    lax.fori_loop(0, epw // LANES, body, 0)
    pltpu.sync_copy(degbuf, out_hbm.at[wid])

  return deg_kernel(dst)


def _sc_scatter_rows(table, src2d, dst2d, n):
  d = table.shape[1]
  nch_total = src2d.shape[0]
  nch = nch_total // NW
  rows_per_tile = n // NS
  zrows = 125
  mesh = plsc.VectorSubcoreMesh(core_axis_name="c", subcore_axis_name="s")

  @functools.partial(
      pl.kernel,
      mesh=mesh,
      out_type=jax.ShapeDtypeStruct((NC, n, d), jnp.float32),
      scratch_types=[
          pltpu.VMEM_SHARED((n, d), jnp.float32),
          pltpu.VMEM((nch, CH), jnp.int32),
          pltpu.VMEM((nch, CH), jnp.int32),
          pltpu.VMEM((CH, d), jnp.float32),
          pltpu.VMEM((125, d), jnp.float32),
          pltpu.SemaphoreType.DMA,
      ],
  )
  def scat_kernel(table_hbm, src_hbm, dst_hbm, out_hbm,
                  acc_sh, srcb, dstb, rowb, zbuf, sem):
    cid = lax.axis_index("c")
    sid = lax.axis_index("s")
    wid = sid * NC + cid

    zero = jnp.zeros((LANES,), jnp.float32)

    def zbody(r, carry):
      for c in range(d // LANES):
        zbuf[r, pl.ds(c * LANES, LANES)] = zero
      return carry

    lax.fori_loop(0, zrows, zbody, 0)

    row0 = sid * rows_per_tile
    for k in range(rows_per_tile // zrows):
      pltpu.sync_copy(zbuf, acc_sh.at[pl.ds(row0 + k * zrows, zrows)])
    plsc.subcore_barrier()

    pltpu.sync_copy(src_hbm.at[pl.ds(wid * nch, nch)], srcb)
    pltpu.sync_copy(dst_hbm.at[pl.ds(wid * nch, nch)], dstb)

    def body(c, carry):
      pltpu.async_copy(table_hbm.at[srcb.at[c]], rowb, sem).wait()
      pltpu.sync_copy(rowb, acc_sh.at[dstb.at[c]], add=True)
      return carry

    lax.fori_loop(0, nch, body, 0)
    plsc.subcore_barrier()

    pltpu.sync_copy(acc_sh.at[pl.ds(row0, rows_per_tile)],
                    out_hbm.at[cid, pl.ds(row0, rows_per_tile)])

  zrows = 125
  return scat_kernel(table, src2d, dst2d)


def _tc1(x, w1, disb, n, blk=1000):
  d = w1.shape[1]

  def body(x_ref, w_ref, disb_ref, g1_ref, h1s_ref):
    g = jnp.dot(x_ref[...], w_ref[...], preferred_element_type=jnp.float32)
    g1_ref[...] = g
    h1s_ref[...] = g * disb_ref[...]

  return pl.pallas_call(
      body,
      grid=(n // blk,),
      in_specs=[
          pl.BlockSpec((blk, x.shape[1]), lambda i: (i, 0)),
          pl.BlockSpec(w1.shape, lambda i: (0, 0)),
          pl.BlockSpec((blk, d), lambda i: (i, 0)),
      ],
      out_specs=[
          pl.BlockSpec((blk, d), lambda i: (i, 0)),
          pl.BlockSpec((blk, d), lambda i: (i, 0)),
      ],
      out_shape=[
          jax.ShapeDtypeStruct((n, d), jnp.float32),
          jax.ShapeDtypeStruct((n, d), jnp.float32),
      ],
  )(x, w1, disb)


def _tc2(s1a, s1b, g1, disb, b1, w2, n, blk=1000):
  d = w2.shape[1]

  def body(sa_ref, sb_ref, g1_ref, disb_ref, b1_ref, w2_ref, g2_ref, h2s_ref):
    dis = disb_ref[...]
    h = jnp.maximum(
        dis * (sa_ref[...] + sb_ref[...]) + dis * dis * g1_ref[...]
        + b1_ref[...], 0.0)
    g2 = jnp.dot(h, w2_ref[...], preferred_element_type=jnp.float32)
    g2_ref[...] = g2
    h2s_ref[...] = g2 * dis

  return pl.pallas_call(
      body,
      grid=(n // blk,),
      in_specs=[
          pl.BlockSpec((blk, d), lambda i: (i, 0)),
          pl.BlockSpec((blk, d), lambda i: (i, 0)),
          pl.BlockSpec((blk, d), lambda i: (i, 0)),
          pl.BlockSpec((blk, d), lambda i: (i, 0)),
          pl.BlockSpec((1, d), lambda i: (0, 0)),
          pl.BlockSpec(w2.shape, lambda i: (0, 0)),
      ],
      out_specs=[
          pl.BlockSpec((blk, d), lambda i: (i, 0)),
          pl.BlockSpec((blk, d), lambda i: (i, 0)),
      ],
      out_shape=[
          jax.ShapeDtypeStruct((n, d), jnp.float32),
          jax.ShapeDtypeStruct((n, d), jnp.float32),
      ],
  )(s1a, s1b, g1, disb, b1, w2)


def _tc3(s2a, s2b, g2, disb, b2, x, wp1a, wp1b, bp1, wp2, bp2, n, blk=1000):
  d = g2.shape[1]
  din = x.shape[1]
  pout = wp2.shape[1]

  def body(sa_ref, sb_ref, g2_ref, disb_ref, b2_ref, x_ref,
           wa_ref, wb_ref, bp1_ref, wp2_ref, bp2_ref, y_ref):
    dis = disb_ref[...]
    gnn = (dis * (sa_ref[...] + sb_ref[...]) + dis * dis * g2_ref[...]
           + b2_ref[...])
    p = jnp.maximum(
        jnp.dot(gnn, wa_ref[...], preferred_element_type=jnp.float32)
        + jnp.dot(x_ref[...], wb_ref[...], preferred_element_type=jnp.float32)
        + bp1_ref[...], 0.0)
    y_ref[...] = (jnp.dot(p, wp2_ref[...], preferred_element_type=jnp.float32)
                  + bp2_ref[...])

  return pl.pallas_call(
      body,
      grid=(n // blk,),
      in_specs=[
          pl.BlockSpec((blk, d), lambda i: (i, 0)),
          pl.BlockSpec((blk, d), lambda i: (i, 0)),
          pl.BlockSpec((blk, d), lambda i: (i, 0)),
          pl.BlockSpec((blk, d), lambda i: (i, 0)),
          pl.BlockSpec((1, d), lambda i: (0, 0)),
          pl.BlockSpec((blk, din), lambda i: (i, 0)),
          pl.BlockSpec(wp1a.shape, lambda i: (0, 0)),
          pl.BlockSpec(wp1b.shape, lambda i: (0, 0)),
          pl.BlockSpec((1, wp1a.shape[1]), lambda i: (0, 0)),
          pl.BlockSpec(wp2.shape, lambda i: (0, 0)),
          pl.BlockSpec((1, pout), lambda i: (0, 0)),
      ],
      out_specs=pl.BlockSpec((blk, pout), lambda i: (i, 0)),
      out_shape=jax.ShapeDtypeStruct((n, pout), jnp.float32),
  )(s2a, s2b, g2, disb, b2, x, wp1a, wp1b, bp1, wp2, bp2)


def kernel(x, edge_index, W1, b1, W2, b2, Wp1, bp1, Wp2, bp2):
  n, din = x.shape
  e = edge_index.shape[1]
  d = W1.shape[1]

  src2d = edge_index[0].reshape(e // CH, CH)
  dst2d = edge_index[1].reshape(e // CH, CH)

  deg_parts = _sc_degree(edge_index[1], n)
  dis = lax.rsqrt(1.0 + jnp.sum(deg_parts, axis=0))
  disb = jnp.broadcast_to(dis[:, None], (n, d))

  g1, h1s = _tc1(x, W1, disb, n)
  s1 = _sc_scatter_rows(h1s, src2d, dst2d, n)
  g2, h2s = _tc2(s1[0], s1[1], g1, disb, b1.reshape(1, d), W2, n)
  s2 = _sc_scatter_rows(h2s, src2d, dst2d, n)
  y = _tc3(s2[0], s2[1], g2, disb, b2.reshape(1, d), x,
           Wp1[:d], Wp1[d:], bp1.reshape(1, -1), Wp2, bp2.reshape(1, -1), n)
  return y


# speedup vs baseline: 17.7406x; 17.7406x over previous
import functools

import jax
import jax.numpy as jnp
from jax import lax
from jax.experimental import pallas as pl
from jax.experimental.pallas import tpu as pltpu
from jax.experimental.pallas import tpu_sc as plsc

NC = 2
NS = 16
NW = NC * NS
LANES = 16
CH = 80


def _sc_degree(dst, n):
  e = dst.shape[0]
  epw = e // NW
  mesh = plsc.VectorSubcoreMesh(core_axis_name="c", subcore_axis_name="s")

  @functools.partial(
      pl.kernel,
      mesh=mesh,
      out_type=jax.ShapeDtypeStruct((NW, n), jnp.float32),
      compiler_params=pltpu.CompilerParams(needs_layout_passes=False, use_tc_tiling_on_sc=False),
      scratch_types=[
          pltpu.VMEM((n,), jnp.float32),
          pltpu.VMEM((epw,), jnp.int32),
      ],
  )
  def deg_kernel(dst_hbm, out_hbm, degbuf, dstbuf):
    cid = lax.axis_index("c")
    sid = lax.axis_index("s")
    wid = sid * NC + cid

    zero = jnp.zeros((LANES,), jnp.float32)

    def zbody(i, carry):
      degbuf[pl.ds(i * LANES, LANES)] = zero
      return carry

    lax.fori_loop(0, n // LANES, zbody, 0)

    pltpu.sync_copy(dst_hbm.at[pl.ds(wid * epw, epw)], dstbuf)

    ones = jnp.ones((LANES,), jnp.float32)

    def body(j, carry):
      idx = dstbuf[pl.ds(j * LANES, LANES)]
      plsc.addupdate_scatter(degbuf, [idx], ones)
      return carry

    lax.fori_loop(0, epw // LANES, body, 0)
    pltpu.sync_copy(degbuf, out_hbm.at[wid])

  return deg_kernel(dst)


def _sc_scatter_rows(table, src2d, dst2d, n):
  d = table.shape[1]
  nch_total = src2d.shape[0]
  nch = nch_total // NW
  rows_per_tile = n // NS
  zrows = 125
  mesh = plsc.VectorSubcoreMesh(core_axis_name="c", subcore_axis_name="s")

  @functools.partial(
      pl.kernel,
      mesh=mesh,
      out_type=jax.ShapeDtypeStruct((NC, n, d), jnp.float32),
      compiler_params=pltpu.CompilerParams(needs_layout_passes=False, use_tc_tiling_on_sc=False),
      scratch_types=[
          pltpu.VMEM_SHARED((n, d), jnp.float32),
          pltpu.VMEM((nch, CH), jnp.int32),
          pltpu.VMEM((nch, CH), jnp.int32),
          pltpu.VMEM((CH, d), jnp.float32),
          pltpu.VMEM((125, d), jnp.float32),
          pltpu.SemaphoreType.DMA,
      ],
  )
  def scat_kernel(table_hbm, src_hbm, dst_hbm, out_hbm,
                  acc_sh, srcb, dstb, rowb, zbuf, sem):
    cid = lax.axis_index("c")
    sid = lax.axis_index("s")
    wid = sid * NC + cid

    zero = jnp.zeros((LANES,), jnp.float32)

    def zbody(r, carry):
      for c in range(d // LANES):
        zbuf[r, pl.ds(c * LANES, LANES)] = zero
      return carry

    lax.fori_loop(0, zrows, zbody, 0)

    row0 = sid * rows_per_tile
    for k in range(rows_per_tile // zrows):
      pltpu.sync_copy(zbuf, acc_sh.at[pl.ds(row0 + k * zrows, zrows)])
    plsc.subcore_barrier()

    pltpu.sync_copy(src_hbm.at[pl.ds(wid * nch, nch)], srcb)
    pltpu.sync_copy(dst_hbm.at[pl.ds(wid * nch, nch)], dstb)

    def body(c, carry):
      pltpu.async_copy(table_hbm.at[srcb.at[c]], rowb, sem).wait()
      pltpu.sync_copy(rowb, acc_sh.at[dstb.at[c]], add=True)
      return carry

    lax.fori_loop(0, nch, body, 0)
    plsc.subcore_barrier()

    pltpu.sync_copy(acc_sh.at[pl.ds(row0, rows_per_tile)],
                    out_hbm.at[cid, pl.ds(row0, rows_per_tile)])

  return scat_kernel(table, src2d, dst2d)


def _tc1(x, w1, disb, n, blk=1000):
  d = w1.shape[1]

  def body(x_ref, w_ref, disb_ref, g1_ref, h1s_ref):
    g = jnp.dot(x_ref[...], w_ref[...], preferred_element_type=jnp.float32)
    g1_ref[...] = g
    h1s_ref[...] = g * disb_ref[...]

  return pl.pallas_call(
      body,
      grid=(n // blk,),
      in_specs=[
          pl.BlockSpec((blk, x.shape[1]), lambda i: (i, 0)),
          pl.BlockSpec(w1.shape, lambda i: (0, 0)),
          pl.BlockSpec((blk, d), lambda i: (i, 0)),
      ],
      out_specs=[
          pl.BlockSpec((blk, d), lambda i: (i, 0)),
          pl.BlockSpec((blk, d), lambda i: (i, 0)),
      ],
      out_shape=[
          jax.ShapeDtypeStruct((n, d), jnp.float32),
          jax.ShapeDtypeStruct((n, d), jnp.float32),
      ],
  )(x, w1, disb)


def _tc2(s1a, s1b, g1, disb, b1, w2, n, blk=1000):
  d = w2.shape[1]

  def body(sa_ref, sb_ref, g1_ref, disb_ref, b1_ref, w2_ref, g2_ref, h2s_ref):
    dis = disb_ref[...]
    h = jnp.maximum(
        dis * (sa_ref[...] + sb_ref[...]) + dis * dis * g1_ref[...]
        + b1_ref[...], 0.0)
    g2 = jnp.dot(h, w2_ref[...], preferred_element_type=jnp.float32)
    g2_ref[...] = g2
    h2s_ref[...] = g2 * dis

  return pl.pallas_call(
      body,
      grid=(n // blk,),
      in_specs=[
          pl.BlockSpec((blk, d), lambda i: (i, 0)),
          pl.BlockSpec((blk, d), lambda i: (i, 0)),
          pl.BlockSpec((blk, d), lambda i: (i, 0)),
          pl.BlockSpec((blk, d), lambda i: (i, 0)),
          pl.BlockSpec((1, d), lambda i: (0, 0)),
          pl.BlockSpec(w2.shape, lambda i: (0, 0)),
      ],
      out_specs=[
          pl.BlockSpec((blk, d), lambda i: (i, 0)),
          pl.BlockSpec((blk, d), lambda i: (i, 0)),
      ],
      out_shape=[
          jax.ShapeDtypeStruct((n, d), jnp.float32),
          jax.ShapeDtypeStruct((n, d), jnp.float32),
      ],
  )(s1a, s1b, g1, disb, b1, w2)


def _tc3(s2a, s2b, g2, disb, b2, x, wp1a, wp1b, bp1, wp2, bp2, n, blk=1000):
  d = g2.shape[1]
  din = x.shape[1]
  pout = wp2.shape[1]

  def body(sa_ref, sb_ref, g2_ref, disb_ref, b2_ref, x_ref,
           wa_ref, wb_ref, bp1_ref, wp2_ref, bp2_ref, y_ref):
    dis = disb_ref[...]
    gnn = (dis * (sa_ref[...] + sb_ref[...]) + dis * dis * g2_ref[...]
           + b2_ref[...])
    p = jnp.maximum(
        jnp.dot(gnn, wa_ref[...], preferred_element_type=jnp.float32)
        + jnp.dot(x_ref[...], wb_ref[...], preferred_element_type=jnp.float32)
        + bp1_ref[...], 0.0)
    y_ref[...] = (jnp.dot(p, wp2_ref[...], preferred_element_type=jnp.float32)
                  + bp2_ref[...])

  return pl.pallas_call(
      body,
      grid=(n // blk,),
      in_specs=[
          pl.BlockSpec((blk, d), lambda i: (i, 0)),
          pl.BlockSpec((blk, d), lambda i: (i, 0)),
          pl.BlockSpec((blk, d), lambda i: (i, 0)),
          pl.BlockSpec((blk, d), lambda i: (i, 0)),
          pl.BlockSpec((1, d), lambda i: (0, 0)),
          pl.BlockSpec((blk, din), lambda i: (i, 0)),
          pl.BlockSpec(wp1a.shape, lambda i: (0, 0)),
          pl.BlockSpec(wp1b.shape, lambda i: (0, 0)),
          pl.BlockSpec((1, wp1a.shape[1]), lambda i: (0, 0)),
          pl.BlockSpec(wp2.shape, lambda i: (0, 0)),
          pl.BlockSpec((1, pout), lambda i: (0, 0)),
      ],
      out_specs=pl.BlockSpec((blk, pout), lambda i: (i, 0)),
      out_shape=jax.ShapeDtypeStruct((n, pout), jnp.float32),
  )(s2a, s2b, g2, disb, b2, x, wp1a, wp1b, bp1, wp2, bp2)


def kernel(x, edge_index, W1, b1, W2, b2, Wp1, bp1, Wp2, bp2):
  n, din = x.shape
  e = edge_index.shape[1]
  d = W1.shape[1]

  src2d = edge_index[0].reshape(e // CH, CH)
  dst2d = edge_index[1].reshape(e // CH, CH)

  deg_parts = _sc_degree(edge_index[1], n)
  dis = lax.rsqrt(1.0 + jnp.sum(deg_parts, axis=0))
  disb = jnp.broadcast_to(dis[:, None], (n, d))

  g1, h1s = _tc1(x, W1, disb, n)
  s1 = _sc_scatter_rows(h1s, src2d, dst2d, n)
  g2, h2s = _tc2(s1[0], s1[1], g1, disb, b1.reshape(1, d), W2, n)
  s2 = _sc_scatter_rows(h2s, src2d, dst2d, n)
  y = _tc3(s2[0], s2[1], g2, disb, b2.reshape(1, d), x,
           Wp1[:d], Wp1[d:], bp1.reshape(1, -1), Wp2, bp2.reshape(1, -1), n)
  return y


# speedup vs baseline: 22.1688x; 1.2496x over previous
import functools

import jax
import jax.numpy as jnp
from jax import lax
from jax.experimental import pallas as pl
from jax.experimental.pallas import tpu as pltpu
from jax.experimental.pallas import tpu_sc as plsc

NC = 2
NS = 16
NW = NC * NS
LANES = 16
CH = 80


def _sc_degree(dst, n):
  e = dst.shape[0]
  epw = e // NW
  mesh = plsc.VectorSubcoreMesh(core_axis_name="c", subcore_axis_name="s")

  @functools.partial(
      pl.kernel,
      mesh=mesh,
      out_type=jax.ShapeDtypeStruct((NW, n), jnp.float32),
      compiler_params=pltpu.CompilerParams(needs_layout_passes=False, use_tc_tiling_on_sc=False),
      scratch_types=[
          pltpu.VMEM((n,), jnp.float32),
          pltpu.VMEM((epw,), jnp.int32),
      ],
  )
  def deg_kernel(dst_hbm, out_hbm, degbuf, dstbuf):
    cid = lax.axis_index("c")
    sid = lax.axis_index("s")
    wid = sid * NC + cid

    zero = jnp.zeros((LANES,), jnp.float32)

    def zbody(i, carry):
      degbuf[pl.ds(i * LANES, LANES)] = zero
      return carry

    lax.fori_loop(0, n // LANES, zbody, 0)

    pltpu.sync_copy(dst_hbm.at[pl.ds(wid * epw, epw)], dstbuf)

    ones = jnp.ones((LANES,), jnp.float32)

    def body(j, carry):
      idx = dstbuf[pl.ds(j * LANES, LANES)]
      plsc.addupdate_scatter(degbuf, [idx], ones)
      return carry

    lax.fori_loop(0, epw // LANES, body, 0)
    pltpu.sync_copy(degbuf, out_hbm.at[wid])

  return deg_kernel(dst)


def _sc_scatter_rows(table, src2d, dst2d, n):
  d = table.shape[1]
  nch_total = src2d.shape[0]
  nch = nch_total // NW
  rows_per_tile = n // NS
  zrows = 25
  mesh = plsc.VectorSubcoreMesh(core_axis_name="c", subcore_axis_name="s")

  @functools.partial(
      pl.kernel,
      mesh=mesh,
      out_type=jax.ShapeDtypeStruct((NC, n, d), jnp.float32),
      compiler_params=pltpu.CompilerParams(needs_layout_passes=False, use_tc_tiling_on_sc=False),
      scratch_types=[
          pltpu.VMEM_SHARED((n, d), jnp.float32),
          pltpu.VMEM((nch, CH), jnp.int32),
          pltpu.VMEM((nch, CH), jnp.int32),
          pltpu.VMEM((2, CH, d), jnp.float32),
          pltpu.VMEM((25, d), jnp.float32),
          pltpu.SemaphoreType.DMA,
          pltpu.SemaphoreType.DMA,
          pltpu.SemaphoreType.DMA,
          pltpu.SemaphoreType.DMA,
      ],
  )
  def scat_kernel(table_hbm, src_hbm, dst_hbm, out_hbm,
                  acc_sh, srcb, dstb, rowb, zbuf, ga, gb, sa, sb):
    cid = lax.axis_index("c")
    sid = lax.axis_index("s")
    wid = sid * NC + cid

    zero = jnp.zeros((LANES,), jnp.float32)

    def zbody(r, carry):
      for c in range(d // LANES):
        zbuf[r, pl.ds(c * LANES, LANES)] = zero
      return carry

    lax.fori_loop(0, zrows, zbody, 0)

    row0 = sid * rows_per_tile
    for k in range(rows_per_tile // zrows):
      pltpu.sync_copy(zbuf, acc_sh.at[pl.ds(row0 + k * zrows, zrows)])
    plsc.subcore_barrier()

    pltpu.sync_copy(src_hbm.at[pl.ds(wid * nch, nch)], srcb)
    pltpu.sync_copy(dst_hbm.at[pl.ds(wid * nch, nch)], dstb)

    bufa = rowb.at[0]
    bufb = rowb.at[1]
    npair = (nch - 1) // 2

    pltpu.async_copy(table_hbm.at[srcb.at[0]], bufa, ga)
    pltpu.async_copy(table_hbm.at[srcb.at[1]], bufb, gb)

    def body(k, carry):
      c = 2 * k
      pltpu.make_async_copy(table_hbm.at[srcb.at[c]], bufa, ga).wait()
      pltpu.async_copy(bufa, acc_sh.at[dstb.at[c]], sa, add=True)
      pltpu.make_async_copy(table_hbm.at[srcb.at[c + 1]], bufb, gb).wait()
      pltpu.async_copy(bufb, acc_sh.at[dstb.at[c + 1]], sb, add=True)
      pltpu.make_async_copy(bufa, acc_sh.at[dstb.at[c]], sa).wait()
      pltpu.async_copy(table_hbm.at[srcb.at[c + 2]], bufa, ga)
      pltpu.make_async_copy(bufb, acc_sh.at[dstb.at[c + 1]], sb).wait()

      @pl.when(k < npair - 1)
      def _():
        pltpu.async_copy(table_hbm.at[srcb.at[c + 3]], bufb, gb)

      return carry

    lax.fori_loop(0, npair, body, 0)
    last = nch - 1
    pltpu.make_async_copy(table_hbm.at[srcb.at[last]], bufa, ga).wait()
    pltpu.async_copy(bufa, acc_sh.at[dstb.at[last]], sa, add=True)
    pltpu.make_async_copy(bufa, acc_sh.at[dstb.at[last]], sa).wait()
    plsc.subcore_barrier()

    pltpu.sync_copy(acc_sh.at[pl.ds(row0, rows_per_tile)],
                    out_hbm.at[cid, pl.ds(row0, rows_per_tile)])

  return scat_kernel(table, src2d, dst2d)


def _tc1(x, w1, disb, n, blk=1000):
  d = w1.shape[1]

  def body(x_ref, w_ref, disb_ref, g1_ref, h1s_ref):
    g = jnp.dot(x_ref[...], w_ref[...], preferred_element_type=jnp.float32)
    g1_ref[...] = g
    h1s_ref[...] = g * disb_ref[...]

  return pl.pallas_call(
      body,
      grid=(n // blk,),
      in_specs=[
          pl.BlockSpec((blk, x.shape[1]), lambda i: (i, 0)),
          pl.BlockSpec(w1.shape, lambda i: (0, 0)),
          pl.BlockSpec((blk, d), lambda i: (i, 0)),
      ],
      out_specs=[
          pl.BlockSpec((blk, d), lambda i: (i, 0)),
          pl.BlockSpec((blk, d), lambda i: (i, 0)),
      ],
      out_shape=[
          jax.ShapeDtypeStruct((n, d), jnp.float32),
          jax.ShapeDtypeStruct((n, d), jnp.float32),
      ],
  )(x, w1, disb)


def _tc2(s1a, s1b, g1, disb, b1, w2, n, blk=1000):
  d = w2.shape[1]

  def body(sa_ref, sb_ref, g1_ref, disb_ref, b1_ref, w2_ref, g2_ref, h2s_ref):
    dis = disb_ref[...]
    h = jnp.maximum(
        dis * (sa_ref[...] + sb_ref[...]) + dis * dis * g1_ref[...]
        + b1_ref[...], 0.0)
    g2 = jnp.dot(h, w2_ref[...], preferred_element_type=jnp.float32)
    g2_ref[...] = g2
    h2s_ref[...] = g2 * dis

  return pl.pallas_call(
      body,
      grid=(n // blk,),
      in_specs=[
          pl.BlockSpec((blk, d), lambda i: (i, 0)),
          pl.BlockSpec((blk, d), lambda i: (i, 0)),
          pl.BlockSpec((blk, d), lambda i: (i, 0)),
          pl.BlockSpec((blk, d), lambda i: (i, 0)),
          pl.BlockSpec((1, d), lambda i: (0, 0)),
          pl.BlockSpec(w2.shape, lambda i: (0, 0)),
      ],
      out_specs=[
          pl.BlockSpec((blk, d), lambda i: (i, 0)),
          pl.BlockSpec((blk, d), lambda i: (i, 0)),
      ],
      out_shape=[
          jax.ShapeDtypeStruct((n, d), jnp.float32),
          jax.ShapeDtypeStruct((n, d), jnp.float32),
      ],
  )(s1a, s1b, g1, disb, b1, w2)


def _tc3(s2a, s2b, g2, disb, b2, x, wp1a, wp1b, bp1, wp2, bp2, n, blk=1000):
  d = g2.shape[1]
  din = x.shape[1]
  pout = wp2.shape[1]

  def body(sa_ref, sb_ref, g2_ref, disb_ref, b2_ref, x_ref,
           wa_ref, wb_ref, bp1_ref, wp2_ref, bp2_ref, y_ref):
    dis = disb_ref[...]
    gnn = (dis * (sa_ref[...] + sb_ref[...]) + dis * dis * g2_ref[...]
           + b2_ref[...])
    p = jnp.maximum(
        jnp.dot(gnn, wa_ref[...], preferred_element_type=jnp.float32)
        + jnp.dot(x_ref[...], wb_ref[...], preferred_element_type=jnp.float32)
        + bp1_ref[...], 0.0)
    y_ref[...] = (jnp.dot(p, wp2_ref[...], preferred_element_type=jnp.float32)
                  + bp2_ref[...])

  return pl.pallas_call(
      body,
      grid=(n // blk,),
      in_specs=[
          pl.BlockSpec((blk, d), lambda i: (i, 0)),
          pl.BlockSpec((blk, d), lambda i: (i, 0)),
          pl.BlockSpec((blk, d), lambda i: (i, 0)),
          pl.BlockSpec((blk, d), lambda i: (i, 0)),
          pl.BlockSpec((1, d), lambda i: (0, 0)),
          pl.BlockSpec((blk, din), lambda i: (i, 0)),
          pl.BlockSpec(wp1a.shape, lambda i: (0, 0)),
          pl.BlockSpec(wp1b.shape, lambda i: (0, 0)),
          pl.BlockSpec((1, wp1a.shape[1]), lambda i: (0, 0)),
          pl.BlockSpec(wp2.shape, lambda i: (0, 0)),
          pl.BlockSpec((1, pout), lambda i: (0, 0)),
      ],
      out_specs=pl.BlockSpec((blk, pout), lambda i: (i, 0)),
      out_shape=jax.ShapeDtypeStruct((n, pout), jnp.float32),
  )(s2a, s2b, g2, disb, b2, x, wp1a, wp1b, bp1, wp2, bp2)


def kernel(x, edge_index, W1, b1, W2, b2, Wp1, bp1, Wp2, bp2):
  n, din = x.shape
  e = edge_index.shape[1]
  d = W1.shape[1]

  src2d = edge_index[0].reshape(e // CH, CH)
  dst2d = edge_index[1].reshape(e // CH, CH)

  deg_parts = _sc_degree(edge_index[1], n)
  dis = lax.rsqrt(1.0 + jnp.sum(deg_parts, axis=0))
  disb = jnp.broadcast_to(dis[:, None], (n, d))

  g1, h1s = _tc1(x, W1, disb, n)
  s1 = _sc_scatter_rows(h1s, src2d, dst2d, n)
  g2, h2s = _tc2(s1[0], s1[1], g1, disb, b1.reshape(1, d), W2, n)
  s2 = _sc_scatter_rows(h2s, src2d, dst2d, n)
  y = _tc3(s2[0], s2[1], g2, disb, b2.reshape(1, d), x,
           Wp1[:d], Wp1[d:], bp1.reshape(1, -1), Wp2, bp2.reshape(1, -1), n)
  return y
